# baseline (speedup 1.0000x reference)
import functools

import jax
import jax.numpy as jnp
from jax import lax
from jax.experimental import pallas as pl
from jax.experimental.pallas import tpu as pltpu
from jax.experimental.pallas import tpu_sc as plsc

N = 10000
E = 320000
D = 128
H = 128
OUT = 10
G = 128

NC = 2
NS = 16
NW = NC * NS
E_T = E // NW
NPAD = 10240
CHUNK = 80
NGRP = CHUNK // 16
NCHUNK = E_T // CHUNK
ROWS_T = NPAD // NS

_MESH = dict(core_axis_name="c", subcore_axis_name="s", num_cores=NC,
             num_subcores=NS)

_BCAST_DN = lax.GatherDimensionNumbers(
    offset_dims=(), collapsed_slice_dims=(0,), start_index_map=(0,))


@functools.partial(
    pl.kernel,
    out_type=jax.ShapeDtypeStruct((NW * NPAD,), jnp.float32),
    mesh=plsc.VectorSubcoreMesh(**_MESH),
    compiler_params=pltpu.CompilerParams(needs_layout_passes=False),
    scratch_types=[
        pltpu.VMEM((E_T,), jnp.int32),
        pltpu.VMEM((E_T,), jnp.float32),
        pltpu.VMEM((NPAD,), jnp.float32),
    ],
)
def _deg(dst_hbm, ew_hbm, out_hbm, dst_v, ew_v, deg_v):
    wid = lax.axis_index("s") * NC + lax.axis_index("c")
    base = wid * E_T
    pltpu.sync_copy(dst_hbm.at[pl.ds(base, E_T)], dst_v)
    pltpu.sync_copy(ew_hbm.at[pl.ds(base, E_T)], ew_v)
    zero = jnp.zeros((16,), jnp.float32)

    def zbody(i, carry):
        deg_v[pl.ds(i * 16, 16)] = zero
        return carry

    lax.fori_loop(0, NPAD // 16, zbody, 0)

    def body(i, carry):
        o = i * 16
        idx = dst_v[pl.ds(o, 16)]
        w = ew_v[pl.ds(o, 16)]
        plsc.addupdate_scatter(deg_v, [idx], w)
        return carry

    lax.fori_loop(0, E_T // 16, body, 0)
    pltpu.sync_copy(deg_v, out_hbm.at[pl.ds(wid * NPAD, NPAD)])


def _dis_body(part_ref, dis_ref, dis2_ref):
    deg = jnp.sum(part_ref[...], axis=0) + 1.0
    dis_ref[...] = lax.rsqrt(deg)
    dis2_ref[...] = 1.0 / deg


_dis = pl.pallas_call(
    _dis_body,
    out_shape=(jax.ShapeDtypeStruct((NPAD,), jnp.float32),
               jax.ShapeDtypeStruct((NPAD,), jnp.float32)),
)


def _make_agg(compute_norm):
    outs = [jax.ShapeDtypeStruct((NC, NPAD, D), jnp.float32)]
    if compute_norm:
        outs.append(jax.ShapeDtypeStruct((E,), jnp.float32))
    scratch = [
        pltpu.VMEM((3 * CHUNK, D), jnp.float32),
        pltpu.VMEM((3, CHUNK), jnp.int32),
        pltpu.VMEM((3, CHUNK), jnp.int32),
        pltpu.VMEM((3, CHUNK), jnp.float32),
        pltpu.VMEM((3, CHUNK), jnp.int32),
        pltpu.SemaphoreType.DMA,
        pltpu.SemaphoreType.DMA,
        pltpu.SemaphoreType.DMA,
        pltpu.SemaphoreType.DMA,
        pltpu.SemaphoreType.DMA,
        pltpu.SemaphoreType.DMA,
        pltpu.SemaphoreType.DMA,
        pltpu.VMEM_SHARED((NPAD, D), jnp.float32),
    ]
    if compute_norm:
        scratch += [
            pltpu.VMEM((NPAD,), jnp.float32),
            pltpu.VMEM((3, CHUNK), jnp.float32),
            pltpu.SemaphoreType.DMA,
        ]

    def body(*refs):
        if compute_norm:
            (x_hbm, src_hbm, dst_hbm, ew_hbm, dis_hbm,
             agg_hbm, nrm_hbm,
             rows_v, sidx_v, cidx_v, wbuf_v, scidx_v,
             gsem0, gsem1, gsem2, ssem0, ssem1, ssem2, isem, acc_sh,
             dis_v, nout_v, wsem) = refs
        else:
            (x_hbm, src_hbm, dst_hbm, nrm_hbm_in,
             agg_hbm,
             rows_v, sidx_v, cidx_v, wbuf_v, scidx_v,
             gsem0, gsem1, gsem2, ssem0, ssem1, ssem2, isem, acc_sh) = refs
        cid = lax.axis_index("c")
        sid = lax.axis_index("s")
        wid = sid * NC + cid
        base = wid * E_T
        gsems = (gsem0, gsem1, gsem2)
        ssems = (ssem0, ssem1, ssem2)
        whbm = ew_hbm if compute_norm else nrm_hbm_in

        def rows_slot(k):
            return rows_v.at[pl.ds(k * CHUNK, CHUNK)]

        def gather(k):
            pltpu.async_copy(x_hbm.at[sidx_v.at[k]], rows_slot(k), gsems[k])

        def gather_wait(k):
            pltpu.make_async_copy(x_hbm.at[sidx_v.at[k]], rows_slot(k),
                                  gsems[k]).wait()

        def scatter(k):
            pltpu.async_copy(rows_slot(k), acc_sh.at[scidx_v.at[k]],
                             ssems[k], add=True)

        def scatter_wait(k):
            pltpu.make_async_copy(rows_slot(k), acc_sh.at[scidx_v.at[k]],
                                  ssems[k]).wait()

        def idx_prefetch(k, c):
            nco = base + c * CHUNK
            pltpu.async_copy(src_hbm.at[pl.ds(nco, CHUNK)], sidx_v.at[k], isem)
            pltpu.async_copy(dst_hbm.at[pl.ds(nco, CHUNK)], cidx_v.at[k], isem)
            pltpu.async_copy(whbm.at[pl.ds(nco, CHUNK)], wbuf_v.at[k], isem)

        def idx_sync(k, c):
            nco = base + c * CHUNK
            pltpu.sync_copy(src_hbm.at[pl.ds(nco, CHUNK)], sidx_v.at[k])
            pltpu.sync_copy(dst_hbm.at[pl.ds(nco, CHUNK)], cidx_v.at[k])
            pltpu.sync_copy(whbm.at[pl.ds(nco, CHUNK)], wbuf_v.at[k])

        def idx_wait(k):
            pltpu.make_async_copy(src_hbm.at[pl.ds(base, CHUNK)],
                                  sidx_v.at[k], isem).wait()
            pltpu.make_async_copy(dst_hbm.at[pl.ds(base, CHUNK)],
                                  cidx_v.at[k], isem).wait()
            pltpu.make_async_copy(whbm.at[pl.ds(base, CHUNK)],
                                  wbuf_v.at[k], isem).wait()

        if compute_norm:
            pltpu.sync_copy(dis_hbm, dis_v)

        zero = jnp.zeros((16,), jnp.float32)
        for e in range(CHUNK):
            for j in range(D // 16):
                rows_v[e, pl.ds(j * 16, 16)] = zero
        r0 = sid * ROWS_T
        for k in range(ROWS_T // CHUNK):
            pltpu.sync_copy(rows_v.at[pl.ds(0, CHUNK)],
                            acc_sh.at[pl.ds(r0 + k * CHUNK, CHUNK)])
        plsc.subcore_barrier()

        idx_sync(0, 0)
        idx_sync(1, 1)
        gather(0)
        gather(1)

        def chunk_body(ci, carry):
            k = ci % 3
            kn2 = (ci + 2) % 3
            has2 = ci <= NCHUNK - 3

            for j in range(3):
                @pl.when(jnp.logical_and(has2, kn2 == j))
                def _(j=j):
                    idx_prefetch(j, ci + 2)

            c16s = []
            for g in range(NGRP):
                gs = pl.ds(g * 16, 16)
                d16 = cidx_v[k, gs]
                scidx_v[k, gs] = d16
                if compute_norm:
                    s16 = sidx_v[k, gs]
                    w16 = wbuf_v[k, gs]
                    c16 = (plsc.load_gather(dis_v, [s16]) * w16 *
                           plsc.load_gather(dis_v, [d16]))
                    nout_v[k, gs] = c16
                else:
                    c16 = wbuf_v[k, gs]
                c16s.append(c16)

            if compute_norm:
                for j in range(3):
                    @pl.when(k == j)
                    def _(j=j):
                        pltpu.async_copy(
                            nout_v.at[j],
                            nrm_hbm.at[pl.ds(base + ci * CHUNK, CHUNK)],
                            wsem)

                @pl.when(ci >= 2)
                def _():
                    pltpu.make_async_copy(
                        nout_v.at[0], nrm_hbm.at[pl.ds(base, CHUNK)],
                        wsem).wait()

            for j in range(3):
                @pl.when(k == j)
                def _(j=j):
                    gather_wait(j)

            po = k * CHUNK
            for g in range(NGRP):
                c16 = c16s[g]
                for e in range(16):
                    cs = c16[e]
                    row = g * 16 + e
                    for j in range(D // 16):
                        sl = pl.ds(j * 16, 16)
                        rows_v[po + row, sl] = rows_v[po + row, sl] * cs

            for j in range(3):
                @pl.when(jnp.logical_and(
                    jnp.logical_and(has2, ci >= 1), kn2 == j))
                def _(j=j):
                    scatter_wait(j)
            for j in range(3):
                @pl.when(jnp.logical_and(has2, kn2 == j))
                def _(j=j):
                    idx_wait(j)
                    gather(j)

            for j in range(3):
                @pl.when(k == j)
                def _(j=j):
                    scatter(j)

            return carry

        lax.fori_loop(0, NCHUNK, chunk_body, 0)
        scatter_wait(0)
        scatter_wait(1)
        scatter_wait(2)
        if compute_norm:
            pltpu.make_async_copy(nout_v.at[0], nrm_hbm.at[pl.ds(base, CHUNK)],
                                  wsem).wait()
            pltpu.make_async_copy(nout_v.at[0], nrm_hbm.at[pl.ds(base, CHUNK)],
                                  wsem).wait()
        plsc.subcore_barrier()

        pltpu.sync_copy(acc_sh.at[pl.ds(r0, ROWS_T)],
                        agg_hbm.at[cid].at[pl.ds(r0, ROWS_T)])

    return pl.kernel(
        body,
        out_type=tuple(outs) if compute_norm else outs[0],
        mesh=plsc.VectorSubcoreMesh(**_MESH),
        compiler_params=pltpu.CompilerParams(needs_layout_passes=False),
        scratch_types=scratch,
    )


_agg_l1 = _make_agg(True)
_agg_l2 = _make_agg(False)


_BR = 2000


def _mm_body(agg_ref, x_ref, dis2_ref, w_ref, b_ref, out_ref):
    pre = agg_ref[0] + agg_ref[1] + dis2_ref[...] * x_ref[...]
    out_ref[...] = jnp.maximum(
        jnp.dot(pre, w_ref[...], preferred_element_type=jnp.float32)
        + b_ref[...], 0.0)


_mm = pl.pallas_call(
    _mm_body,
    grid=(N // _BR,),
    in_specs=[
        pl.BlockSpec((NC, _BR, D), lambda i: (0, i, 0)),
        pl.BlockSpec((_BR, D), lambda i: (i, 0)),
        pl.BlockSpec((_BR, 1), lambda i: (i, 0)),
        pl.BlockSpec((D, H), lambda i: (0, 0)),
        pl.BlockSpec((1, H), lambda i: (0, 0)),
    ],
    out_specs=pl.BlockSpec((_BR, H), lambda i: (i, 0)),
    out_shape=jax.ShapeDtypeStruct((N, H), jnp.float32),
)


def _final_body(agg_ref, h1_ref, dis2_ref, batch_ref, w2_ref, b2_ref,
                wl1_ref, bl1_ref, wl2_ref, bl2_ref, out_ref, pool_acc):
    i = pl.program_id(0)
    pre = agg_ref[0] + agg_ref[1] + dis2_ref[...] * h1_ref[...]
    h2 = jnp.maximum(
        jnp.dot(pre, w2_ref[...], preferred_element_type=jnp.float32)
        + b2_ref[...], 0.0)
    onehot = (batch_ref[...] ==
              lax.broadcasted_iota(jnp.int32, (_BR, G), 1)).astype(jnp.float32)
    contrib = lax.dot_general(onehot, h2, (((0,), (0,)), ((), ())),
                              preferred_element_type=jnp.float32)

    @pl.when(i == 0)
    def _():
        pool_acc[...] = contrib

    @pl.when(i > 0)
    def _():
        pool_acc[...] += contrib

    @pl.when(i == pl.num_programs(0) - 1)
    def _():
        hh = jnp.maximum(
            jnp.dot(pool_acc[...], wl1_ref[...],
                    preferred_element_type=jnp.float32) + bl1_ref[...], 0.0)
        out_ref[...] = jnp.dot(hh, wl2_ref[...],
                               preferred_element_type=jnp.float32) + bl2_ref[...]


_final = pl.pallas_call(
    _final_body,
    grid=(N // _BR,),
    in_specs=[
        pl.BlockSpec((NC, _BR, D), lambda i: (0, i, 0)),
        pl.BlockSpec((_BR, H), lambda i: (i, 0)),
        pl.BlockSpec((_BR, 1), lambda i: (i, 0)),
        pl.BlockSpec((_BR, 1), lambda i: (i, 0)),
        pl.BlockSpec((H, H), lambda i: (0, 0)),
        pl.BlockSpec((1, H), lambda i: (0, 0)),
        pl.BlockSpec((H, H), lambda i: (0, 0)),
        pl.BlockSpec((1, H), lambda i: (0, 0)),
        pl.BlockSpec((H, H), lambda i: (0, 0)),
        pl.BlockSpec((1, H), lambda i: (0, 0)),
    ],
    out_specs=pl.BlockSpec((G, H), lambda i: (0, 0)),
    out_shape=jax.ShapeDtypeStruct((G, H), jnp.float32),
    scratch_shapes=[pltpu.VMEM((G, H), jnp.float32)],
)


def kernel(x, edge_index, batch, edge_attr, Wg1, bg1, Wg2, bg2,
           Wl1, bl1, Wl2, bl2):
    src = edge_index[0].astype(jnp.int32)
    dst = edge_index[1].astype(jnp.int32)
    ew = edge_attr.astype(jnp.float32)

    degp = _deg(dst, ew).reshape(NW, NPAD)
    dis, dis2 = _dis(degp)
    dis2c = dis2[:N, None]

    agg1, norm = _agg_l1(x, src, dst, ew, dis)
    h1 = _mm(agg1, x, dis2c, Wg1, bg1[None, :])
    agg2 = _agg_l2(h1, src, dst, norm)

    wl2p = jnp.zeros((H, H), jnp.float32).at[:, :OUT].set(Wl2)
    bl2p = jnp.zeros((1, H), jnp.float32).at[0, :OUT].set(bl2)
    outp = _final(agg2, h1, dis2c, batch.astype(jnp.int32)[:, None],
                  Wg2, bg2[None, :], Wl1, bl1[None, :], wl2p, bl2p)
    return outp[:, :OUT]


# speedup vs baseline: 1.0149x; 1.0149x over previous
import functools

import jax
import jax.numpy as jnp
from jax import lax
from jax.experimental import pallas as pl
from jax.experimental.pallas import tpu as pltpu
from jax.experimental.pallas import tpu_sc as plsc

N = 10000
E = 320000
D = 128
H = 128
OUT = 10
G = 128

NC = 2
NS = 16
NW = NC * NS
E_T = E // NW
NPAD = 10240
CHUNK = 80
NGRP = CHUNK // 16
NCHUNK = E_T // CHUNK
ROWS_T = NPAD // NS
BLKC = 5
BLKE = BLKC * CHUNK
NBLK = NCHUNK // BLKC

_MESH = dict(core_axis_name="c", subcore_axis_name="s", num_cores=NC,
             num_subcores=NS)

_BCAST_DN = lax.GatherDimensionNumbers(
    offset_dims=(), collapsed_slice_dims=(0,), start_index_map=(0,))


@functools.partial(
    pl.kernel,
    out_type=jax.ShapeDtypeStruct((NW * NPAD,), jnp.float32),
    mesh=plsc.VectorSubcoreMesh(**_MESH),
    compiler_params=pltpu.CompilerParams(needs_layout_passes=False),
    scratch_types=[
        pltpu.VMEM((E_T,), jnp.int32),
        pltpu.VMEM((E_T,), jnp.float32),
        pltpu.VMEM((NPAD,), jnp.float32),
    ],
)
def _deg(dst_hbm, ew_hbm, out_hbm, dst_v, ew_v, deg_v):
    wid = lax.axis_index("s") * NC + lax.axis_index("c")
    base = wid * E_T
    pltpu.sync_copy(dst_hbm.at[pl.ds(base, E_T)], dst_v)
    pltpu.sync_copy(ew_hbm.at[pl.ds(base, E_T)], ew_v)
    zero = jnp.zeros((16,), jnp.float32)

    def zbody(i, carry):
        deg_v[pl.ds(i * 16, 16)] = zero
        return carry

    lax.fori_loop(0, NPAD // 16, zbody, 0)

    def body(i, carry):
        o = i * 16
        idx = dst_v[pl.ds(o, 16)]
        w = ew_v[pl.ds(o, 16)]
        plsc.addupdate_scatter(deg_v, [idx], w)
        return carry

    lax.fori_loop(0, E_T // 16, body, 0)
    pltpu.sync_copy(deg_v, out_hbm.at[pl.ds(wid * NPAD, NPAD)])


def _dis_body(part_ref, dis_ref, dis2_ref):
    deg = jnp.sum(part_ref[...], axis=0) + 1.0
    dis_ref[...] = lax.rsqrt(deg)
    dis2_ref[...] = 1.0 / deg


_dis = pl.pallas_call(
    _dis_body,
    out_shape=(jax.ShapeDtypeStruct((NPAD,), jnp.float32),
               jax.ShapeDtypeStruct((NPAD,), jnp.float32)),
)


def _make_agg(compute_norm):
    outs = [jax.ShapeDtypeStruct((NC, NPAD, D), jnp.float32)]
    if compute_norm:
        outs.append(jax.ShapeDtypeStruct((E,), jnp.float32))
    scratch = [
        pltpu.VMEM((3 * CHUNK, D), jnp.float32),
        pltpu.VMEM((1024,), jnp.int32),
        pltpu.VMEM((1024,), jnp.int32),
        pltpu.VMEM((1024,), jnp.float32),
        pltpu.VMEM((3, CHUNK), jnp.int32),
        pltpu.SemaphoreType.DMA,
        pltpu.SemaphoreType.DMA,
        pltpu.SemaphoreType.DMA,
        pltpu.SemaphoreType.DMA,
        pltpu.SemaphoreType.DMA,
        pltpu.SemaphoreType.DMA,
        pltpu.SemaphoreType.DMA,
        pltpu.VMEM_SHARED((NPAD, D), jnp.float32),
    ]
    if compute_norm:
        scratch += [
            pltpu.VMEM((NPAD,), jnp.float32),
            pltpu.VMEM((3, CHUNK), jnp.float32),
            pltpu.SemaphoreType.DMA,
        ]

    def body(*refs):
        if compute_norm:
            (x_hbm, src_hbm, dst_hbm, ew_hbm, dis_hbm,
             agg_hbm, nrm_hbm,
             rows_v, sblk_v, dblk_v, wblk_v, scidx_v,
             gsem0, gsem1, gsem2, ssem0, ssem1, ssem2, isem, acc_sh,
             dis_v, nout_v, wsem) = refs
        else:
            (x_hbm, src_hbm, dst_hbm, nrm_hbm_in,
             agg_hbm,
             rows_v, sblk_v, dblk_v, wblk_v, scidx_v,
             gsem0, gsem1, gsem2, ssem0, ssem1, ssem2, isem, acc_sh) = refs
        cid = lax.axis_index("c")
        sid = lax.axis_index("s")
        wid = sid * NC + cid
        base = wid * E_T
        gsems = (gsem0, gsem1, gsem2)
        ssems = (ssem0, ssem1, ssem2)
        whbm = ew_hbm if compute_norm else nrm_hbm_in

        def rows_slot(k):
            return rows_v.at[pl.ds(k * CHUNK, CHUNK)]

        def gather(k, ioff):
            pltpu.async_copy(x_hbm.at[sblk_v.at[pl.ds(ioff, CHUNK)]],
                             rows_slot(k), gsems[k])

        def gather_wait(k):
            pltpu.make_async_copy(x_hbm.at[scidx_v.at[k]], rows_slot(k),
                                  gsems[k]).wait()

        def scatter(k):
            pltpu.async_copy(rows_slot(k), acc_sh.at[scidx_v.at[k]],
                             ssems[k], add=True)

        def scatter_wait(k):
            pltpu.make_async_copy(rows_slot(k), acc_sh.at[scidx_v.at[k]],
                                  ssems[k]).wait()

        def blk_load(b, soff, do_async):
            bo = base + b * BLKE
            dsts = (sblk_v, dblk_v, wblk_v)
            srcs = (src_hbm, dst_hbm, whbm)
            for sr, dv in zip(srcs, dsts):
                if do_async:
                    pltpu.async_copy(sr.at[pl.ds(bo, BLKE)],
                                     dv.at[pl.ds(soff, BLKE)], isem)
                else:
                    pltpu.sync_copy(sr.at[pl.ds(bo, BLKE)],
                                    dv.at[pl.ds(soff, BLKE)])

        def blk_wait():
            for sr, dv in ((src_hbm, sblk_v), (dst_hbm, dblk_v),
                           (whbm, wblk_v)):
                pltpu.make_async_copy(sr.at[pl.ds(base, BLKE)],
                                      dv.at[pl.ds(0, BLKE)], isem).wait()

        if compute_norm:
            pltpu.sync_copy(dis_hbm, dis_v)

        zero = jnp.zeros((16,), jnp.float32)
        for e in range(CHUNK):
            for j in range(D // 16):
                rows_v[e, pl.ds(j * 16, 16)] = zero
        r0 = sid * ROWS_T
        for k in range(ROWS_T // CHUNK):
            pltpu.sync_copy(rows_v.at[pl.ds(0, CHUNK)],
                            acc_sh.at[pl.ds(r0 + k * CHUNK, CHUNK)])
        plsc.subcore_barrier()

        blk_load(0, 0, False)
        blk_load(1, 512, True)
        gather(0, 0)
        gather(1, CHUNK)

        def chunk_body(ci, carry):
            k = ci % 3
            kn2 = (ci + 2) % 3
            has2 = ci <= NCHUNK - 3

            bi = ci // BLKC
            @pl.when(jnp.logical_and(
                jnp.logical_and(ci % BLKC == 0, ci >= BLKC),
                ci < (NBLK - 1) * BLKC))
            def _():
                blk_load(bi + 1, ((bi + 1) % 2) * 512, True)

            boff = (bi % 2) * 512 + (ci % BLKC) * CHUNK
            c16s = []
            for g in range(NGRP):
                gs = pl.ds(g * 16, 16)
                bs = pl.ds(boff + g * 16, 16)
                d16 = dblk_v[bs]
                scidx_v[k, gs] = d16
                if compute_norm:
                    s16 = sblk_v[bs]
                    w16 = wblk_v[bs]
                    c16 = (plsc.load_gather(dis_v, [s16]) * w16 *
                           plsc.load_gather(dis_v, [d16]))
                    nout_v[k, gs] = c16
                else:
                    c16 = wblk_v[bs]
                c16s.append(c16)

            if compute_norm:
                for j in range(3):
                    @pl.when(k == j)
                    def _(j=j):
                        pltpu.async_copy(
                            nout_v.at[j],
                            nrm_hbm.at[pl.ds(base + ci * CHUNK, CHUNK)],
                            wsem)

                @pl.when(ci >= 2)
                def _():
                    pltpu.make_async_copy(
                        nout_v.at[0], nrm_hbm.at[pl.ds(base, CHUNK)],
                        wsem).wait()

            for j in range(3):
                @pl.when(k == j)
                def _(j=j):
                    gather_wait(j)

            po = k * CHUNK
            for g in range(NGRP):
                c16 = c16s[g]
                for e in range(16):
                    cs = c16[e]
                    row = g * 16 + e
                    for j in range(D // 16):
                        sl = pl.ds(j * 16, 16)
                        rows_v[po + row, sl] = rows_v[po + row, sl] * cs

            for j in range(3):
                @pl.when(jnp.logical_and(
                    jnp.logical_and(has2, ci >= 1), kn2 == j))
                def _(j=j):
                    scatter_wait(j)
            @pl.when(jnp.logical_and(ci % BLKC == BLKC - 2, has2))
            def _():
                blk_wait()
            noff = ((ci + 2) // BLKC % 2) * 512 + ((ci + 2) % BLKC) * CHUNK
            for j in range(3):
                @pl.when(jnp.logical_and(has2, kn2 == j))
                def _(j=j):
                    gather(j, noff)

            for j in range(3):
                @pl.when(k == j)
                def _(j=j):
                    scatter(j)

            return carry

        lax.fori_loop(0, NCHUNK, chunk_body, 0)
        scatter_wait(0)
        scatter_wait(1)
        scatter_wait(2)
        if compute_norm:
            pltpu.make_async_copy(nout_v.at[0], nrm_hbm.at[pl.ds(base, CHUNK)],
                                  wsem).wait()
            pltpu.make_async_copy(nout_v.at[0], nrm_hbm.at[pl.ds(base, CHUNK)],
                                  wsem).wait()
        plsc.subcore_barrier()

        pltpu.sync_copy(acc_sh.at[pl.ds(r0, ROWS_T)],
                        agg_hbm.at[cid].at[pl.ds(r0, ROWS_T)])

    return pl.kernel(
        body,
        out_type=tuple(outs) if compute_norm else outs[0],
        mesh=plsc.VectorSubcoreMesh(**_MESH),
        compiler_params=pltpu.CompilerParams(needs_layout_passes=False),
        scratch_types=scratch,
    )


_agg_l1 = _make_agg(True)
_agg_l2 = _make_agg(False)


_BR = 2000


def _mm_body(agg_ref, x_ref, dis2_ref, w_ref, b_ref, out_ref):
    pre = agg_ref[0] + agg_ref[1] + dis2_ref[...] * x_ref[...]
    out_ref[...] = jnp.maximum(
        jnp.dot(pre, w_ref[...], preferred_element_type=jnp.float32)
        + b_ref[...], 0.0)


_mm = pl.pallas_call(
    _mm_body,
    grid=(N // _BR,),
    in_specs=[
        pl.BlockSpec((NC, _BR, D), lambda i: (0, i, 0)),
        pl.BlockSpec((_BR, D), lambda i: (i, 0)),
        pl.BlockSpec((_BR, 1), lambda i: (i, 0)),
        pl.BlockSpec((D, H), lambda i: (0, 0)),
        pl.BlockSpec((1, H), lambda i: (0, 0)),
    ],
    out_specs=pl.BlockSpec((_BR, H), lambda i: (i, 0)),
    out_shape=jax.ShapeDtypeStruct((N, H), jnp.float32),
)


def _final_body(agg_ref, h1_ref, dis2_ref, batch_ref, w2_ref, b2_ref,
                wl1_ref, bl1_ref, wl2_ref, bl2_ref, out_ref, pool_acc):
    i = pl.program_id(0)
    pre = agg_ref[0] + agg_ref[1] + dis2_ref[...] * h1_ref[...]
    h2 = jnp.maximum(
        jnp.dot(pre, w2_ref[...], preferred_element_type=jnp.float32)
        + b2_ref[...], 0.0)
    onehot = (batch_ref[...] ==
              lax.broadcasted_iota(jnp.int32, (_BR, G), 1)).astype(jnp.float32)
    contrib = lax.dot_general(onehot, h2, (((0,), (0,)), ((), ())),
                              preferred_element_type=jnp.float32)

    @pl.when(i == 0)
    def _():
        pool_acc[...] = contrib

    @pl.when(i > 0)
    def _():
        pool_acc[...] += contrib

    @pl.when(i == pl.num_programs(0) - 1)
    def _():
        hh = jnp.maximum(
            jnp.dot(pool_acc[...], wl1_ref[...],
                    preferred_element_type=jnp.float32) + bl1_ref[...], 0.0)
        out_ref[...] = jnp.dot(hh, wl2_ref[...],
                               preferred_element_type=jnp.float32) + bl2_ref[...]


_final = pl.pallas_call(
    _final_body,
    grid=(N // _BR,),
    in_specs=[
        pl.BlockSpec((NC, _BR, D), lambda i: (0, i, 0)),
        pl.BlockSpec((_BR, H), lambda i: (i, 0)),
        pl.BlockSpec((_BR, 1), lambda i: (i, 0)),
        pl.BlockSpec((_BR, 1), lambda i: (i, 0)),
        pl.BlockSpec((H, H), lambda i: (0, 0)),
        pl.BlockSpec((1, H), lambda i: (0, 0)),
        pl.BlockSpec((H, H), lambda i: (0, 0)),
        pl.BlockSpec((1, H), lambda i: (0, 0)),
        pl.BlockSpec((H, H), lambda i: (0, 0)),
        pl.BlockSpec((1, H), lambda i: (0, 0)),
    ],
    out_specs=pl.BlockSpec((G, H), lambda i: (0, 0)),
    out_shape=jax.ShapeDtypeStruct((G, H), jnp.float32),
    scratch_shapes=[pltpu.VMEM((G, H), jnp.float32)],
)


def kernel(x, edge_index, batch, edge_attr, Wg1, bg1, Wg2, bg2,
           Wl1, bl1, Wl2, bl2):
    src = edge_index[0].astype(jnp.int32)
    dst = edge_index[1].astype(jnp.int32)
    ew = edge_attr.astype(jnp.float32)

    degp = _deg(dst, ew).reshape(NW, NPAD)
    dis, dis2 = _dis(degp)
    dis2c = dis2[:N, None]

    agg1, norm = _agg_l1(x, src, dst, ew, dis)
    h1 = _mm(agg1, x, dis2c, Wg1, bg1[None, :])
    agg2 = _agg_l2(h1, src, dst, norm)

    wl2p = jnp.zeros((H, H), jnp.float32).at[:, :OUT].set(Wl2)
    bl2p = jnp.zeros((1, H), jnp.float32).at[0, :OUT].set(bl2)
    outp = _final(agg2, h1, dis2c, batch.astype(jnp.int32)[:, None],
                  Wg2, bg2[None, :], Wl1, bl1[None, :], wl2p, bl2p)
    return outp[:, :OUT]
